# baseline (speedup 1.0000x reference)
import jax
import jax.numpy as jnp
from jax.experimental import pallas as pl
from jax.experimental.pallas import tpu as pltpu

_CT_COLS = 128
_IND_CHUNK = 128


def _body(s_ref, ct_ref, ind_ref, t_ref, saved_ref, new_ref,
          ct_out, ind_out, saved_out, chunk_smem, sem):
    del saved_ref
    idx = s_ref[0]

    base = pl.multiple_of((idx // _IND_CHUNK) * _IND_CHUNK, _IND_CHUNK)
    cur_cp = pltpu.make_async_copy(
        ind_ref.at[pl.ds(base, _IND_CHUNK)], chunk_smem, sem)
    cur_cp.start()
    cur_cp.wait()
    off = idx - base
    cursor = chunk_smem[off]

    chunk_smem[off] = cursor + 1
    ind_fix = pltpu.make_async_copy(
        chunk_smem, ind_out.at[pl.ds(base, _IND_CHUNK)], sem)
    ind_fix.start()

    cc = idx - (idx // _CT_COLS) * _CT_COLS
    row_i = jax.lax.broadcasted_iota(jnp.int32, ct_ref.shape, 0)
    col_i = jax.lax.broadcasted_iota(jnp.int32, ct_ref.shape, 1)
    ct_out[...] = jnp.where((row_i == cursor) & (col_i == cc),
                            t_ref[0], ct_ref[...])

    saved_out[...] = new_ref[...]

    ind_fix.wait()


def kernel(clicktimes, indices, idx, t, saved, new, save_index):
    idx32 = jnp.asarray(idx, jnp.int32)
    si32 = jnp.asarray(save_index, jnp.int32)
    s = jnp.stack([idx32, si32])
    t_arr = jnp.asarray(t, jnp.float32).reshape(1)

    ct_t = clicktimes.T
    saved_t = saved.transpose(0, 2, 1)
    new_t = new.transpose(0, 2, 1)

    n_clicks = ct_t.shape[0]
    slot_blk = (1,) + saved_t.shape[1:]
    grid_spec = pltpu.PrefetchScalarGridSpec(
        num_scalar_prefetch=1,
        grid=(1,),
        in_specs=[
            pl.BlockSpec((n_clicks, _CT_COLS),
                         lambda i, s: (0, s[0] // _CT_COLS)),
            pl.BlockSpec(memory_space=pltpu.HBM),
            pl.BlockSpec(memory_space=pltpu.SMEM),
            pl.BlockSpec(slot_blk, lambda i, s: (s[1], 0, 0)),
            pl.BlockSpec(slot_blk, lambda i, s: (s[1], 0, 0)),
        ],
        out_specs=[
            pl.BlockSpec((n_clicks, _CT_COLS),
                         lambda i, s: (0, s[0] // _CT_COLS)),
            pl.BlockSpec(memory_space=pltpu.HBM),
            pl.BlockSpec(slot_blk, lambda i, s: (s[1], 0, 0)),
        ],
        scratch_shapes=[
            pltpu.SMEM((_IND_CHUNK,), indices.dtype),
            pltpu.SemaphoreType.DMA,
        ],
    )
    ct_out_t, ind_out, saved_out_t = pl.pallas_call(
        _body,
        grid_spec=grid_spec,
        out_shape=[
            jax.ShapeDtypeStruct(ct_t.shape, ct_t.dtype),
            jax.ShapeDtypeStruct(indices.shape, indices.dtype),
            jax.ShapeDtypeStruct(saved_t.shape, saved_t.dtype),
        ],
        input_output_aliases={1: 0, 2: 1, 4: 2},
    )(s, ct_t, indices, t_arr, saved_t, new_t)

    return (ct_out_t.T, ind_out, saved_out_t.transpose(0, 2, 1),
            save_index + 1)
